# baseline (speedup 1.0000x reference)
import functools

import jax
import jax.numpy as jnp
from jax import lax
from jax.experimental import pallas as pl
from jax.experimental.pallas import tpu as pltpu
from jax.experimental.pallas import tpu_sc as plsc

N_NODES = 10000
N_EDGES = 160000
F_DIM = 256
DE_DIM = 16

NC = 2
NS = 16
NW = NC * NS


_ROWS_BLK = 1000


def _rne16(x):
    b = jax.lax.bitcast_convert_type(x, jnp.int32)
    r = b + 0x7FFF + jax.lax.shift_right_logical(b, 16) % 2
    return jax.lax.shift_right_logical(r, 16)


def _pack2(gate_f32, core_f32):
    return _rne16(gate_f32) | (_rne16(core_f32) << 16)


def _unpack_lo(w):
    return jax.lax.bitcast_convert_type(w << 16, jnp.float32)


def _unpack_hi(w):
    return jax.lax.bitcast_convert_type(w & jnp.int32(-65536), jnp.float32)


def _tables_body(x_ref, wfd_ref, wsd_ref, wfs_ref, wss_ref, d_ref, s_ref):
    xb = x_ref[...]
    d_ref[...] = _pack2(
        jnp.dot(xb, wfd_ref[...], preferred_element_type=jnp.float32),
        jnp.dot(xb, wsd_ref[...], preferred_element_type=jnp.float32))
    s_ref[...] = _pack2(
        jnp.dot(xb, wfs_ref[...], preferred_element_type=jnp.float32),
        jnp.dot(xb, wss_ref[...], preferred_element_type=jnp.float32))


def _node_tables(x, wfd, wsd, wfs, wss):
    grid = (N_NODES // _ROWS_BLK,)
    wspec = pl.BlockSpec((F_DIM, F_DIM), lambda i: (0, 0))
    return pl.pallas_call(
        _tables_body,
        grid=grid,
        in_specs=[
            pl.BlockSpec((_ROWS_BLK, F_DIM), lambda i: (i, 0)),
            wspec, wspec, wspec, wspec,
        ],
        out_specs=[
            pl.BlockSpec((_ROWS_BLK, F_DIM), lambda i: (i, 0)),
            pl.BlockSpec((_ROWS_BLK, F_DIM), lambda i: (i, 0)),
        ],
        out_shape=[
            jax.ShapeDtypeStruct((N_NODES, F_DIM), jnp.int32),
            jax.ShapeDtypeStruct((N_NODES, F_DIM), jnp.int32),
        ],
    )(x, wfd, wsd, wfs, wss)


def _make_edge_gather(n_edges, gch):
    ew = n_edges // NS
    git = ew // gch

    def _gather_kernel(d_hbm, s_hbm, dst_hbm, src_hbm, rd_hbm, rs_hbm,
                       idxv, buf0, buf1, sem0, sem1):
        c = lax.axis_index("c")
        s = lax.axis_index("s")
        base = s * ew
        bufs = (buf0, buf1)
        sems = (sem0, sem1)

        def run(tab_hbm, idx_hbm, out_hbm):
            pltpu.sync_copy(idx_hbm.at[pl.ds(base, ew)], idxv)

            def start(b, i):
                pltpu.async_copy(tab_hbm.at[idxv.at[pl.ds(i * gch, gch)]],
                                 bufs[b], sems[b])

            def finish(b, i):
                pltpu.make_async_copy(
                    tab_hbm.at[idxv.at[pl.ds(i * gch, gch)]],
                    bufs[b], sems[b]).wait()
                pltpu.sync_copy(bufs[b],
                                out_hbm.at[pl.ds(base + i * gch, gch)])

            start(0, 0)
            start(1, 1)

            def body(g, carry):
                for b in range(2):
                    i = 2 * g + b
                    finish(b, i)

                    @pl.when(i + 2 < git)
                    def _():
                        start(b, i + 2)
                return carry

            lax.fori_loop(0, git // 2, body, 0)
            if git % 2:
                finish(0, git - 1)

        @pl.when(c == 0)
        def _():
            run(d_hbm, dst_hbm, rd_hbm)

        @pl.when(c == 1)
        def _():
            run(s_hbm, src_hbm, rs_hbm)

    def call(d, s, dst, src):
        k = functools.partial(
            pl.kernel,
            mesh=plsc.VectorSubcoreMesh(core_axis_name="c",
                                        subcore_axis_name="s"),
            out_type=[
                jax.ShapeDtypeStruct((n_edges, F_DIM), jnp.int32),
                jax.ShapeDtypeStruct((n_edges, F_DIM), jnp.int32),
            ],
            scratch_types=[
                pltpu.VMEM((ew,), jnp.int32),
                pltpu.VMEM((gch, F_DIM), jnp.int32),
                pltpu.VMEM((gch, F_DIM), jnp.int32),
                pltpu.SemaphoreType.DMA,
                pltpu.SemaphoreType.DMA,
            ],
        )(_gather_kernel)
        return k(d, s, dst, src)

    return call


_EHALF = N_EDGES // 2
_edge_gather_full = _make_edge_gather(N_EDGES, 80)


_EDGE_BLK = 2000


def _msg_body(rd_ref, rs_ref, ea_ref, weg_ref, wec_ref, bg_ref, bc_ref,
              m0_ref, m1_ref):
    wd = rd_ref[...]
    ws = rs_ref[...]
    ea = ea_ref[...]
    g = (_unpack_lo(wd) + _unpack_lo(ws)
         + jnp.dot(ea, weg_ref[...], preferred_element_type=jnp.float32)
         + bg_ref[...])
    c = (_unpack_hi(wd) + _unpack_hi(ws)
         + jnp.dot(ea, wec_ref[...], preferred_element_type=jnp.float32)
         + bc_ref[...])
    gate = 1.0 / (1.0 + jnp.exp(-g))
    sp = jnp.maximum(c, 0.0) + jnp.log(1.0 + jnp.exp(-jnp.abs(c)))
    msg = gate * sp
    m0_ref[...] = msg[:, : F_DIM // 2]
    m1_ref[...] = msg[:, F_DIM // 2:]


def _edge_messages_half(rd, rs, ea, weg, wec, bg, bc, half):
    grid = (_EHALF // _EDGE_BLK,)
    off = half * (_EHALF // _EDGE_BLK)
    wspec = pl.BlockSpec((DE_DIM, F_DIM), lambda i: (0, 0))
    bspec = pl.BlockSpec((1, F_DIM), lambda i: (0, 0))
    return pl.pallas_call(
        _msg_body,
        grid=grid,
        in_specs=[
            pl.BlockSpec((_EDGE_BLK, F_DIM), lambda i: (i + off, 0)),
            pl.BlockSpec((_EDGE_BLK, F_DIM), lambda i: (i + off, 0)),
            pl.BlockSpec((_EDGE_BLK, DE_DIM), lambda i: (i + off, 0)),
            wspec, wspec, bspec, bspec,
        ],
        out_specs=[
            pl.BlockSpec((_EDGE_BLK, F_DIM // 2), lambda i: (i, 0)),
            pl.BlockSpec((_EDGE_BLK, F_DIM // 2), lambda i: (i, 0)),
        ],
        out_shape=[
            jax.ShapeDtypeStruct((_EHALF, F_DIM // 2), jnp.float32),
            jax.ShapeDtypeStruct((_EHALF, F_DIM // 2), jnp.float32),
        ],
    )(rd, rs, ea, weg, wec, bg, bc)


_ZROWS = 1000
_ZTILES = N_NODES // _ZROWS
_HF = F_DIM // 2


def _make_scatter_add(n_edges, sch):
    et = n_edges // NS
    sit = et // sch

    def _scatter_kernel(m0_hbm, m1_hbm, dst3d_hbm, z0_hbm, z1_hbm,
                        a0_hbm, a1_hbm,
                        idxv, mbuf0, mbuf1, acc_sh,
                        seml0, seml1, semc0, semc1):
        c = lax.axis_index("c")
        s = lax.axis_index("s")

        @pl.when(jnp.logical_and(c == 0, s < _ZTILES))
        def _():
            pltpu.sync_copy(z0_hbm.at[pl.ds(s * _ZROWS, _ZROWS)],
                            acc_sh.at[pl.ds(s * _ZROWS, _ZROWS)])

        @pl.when(jnp.logical_and(c == 1, s < _ZTILES))
        def _():
            pltpu.sync_copy(z1_hbm.at[pl.ds(s * _ZROWS, _ZROWS)],
                            acc_sh.at[pl.ds(s * _ZROWS, _ZROWS)])

        plsc.subcore_barrier()
        pltpu.sync_copy(dst3d_hbm.at[s], idxv)

        mbufs = (mbuf0, mbuf1)
        semls = (seml0, seml1)
        semcs = (semc0, semc1)

        def run(m_hbm):
            def startload(b, i):
                pltpu.async_copy(m_hbm.at[pl.ds(s * et + i * sch, sch)],
                                 mbufs[b], semls[b])

            def finish(b, i):
                pltpu.make_async_copy(
                    m_hbm.at[pl.ds(s * et + i * sch, sch)],
                    mbufs[b], semls[b]).wait()
                pltpu.async_copy(mbufs[b], acc_sh.at[idxv.at[i]], semcs[b],
                                 add=True)
                pltpu.make_async_copy(mbufs[b], acc_sh.at[idxv.at[i]],
                                      semcs[b]).wait()

            startload(0, 0)
            startload(1, 1)

            def body(g, carry):
                for b in range(2):
                    i = 2 * g + b
                    finish(b, i)

                    @pl.when(i + 2 < sit)
                    def _():
                        startload(b, i + 2)
                return carry

            lax.fori_loop(0, sit // 2, body, 0)
            if sit % 2:
                finish(0, sit - 1)

        @pl.when(c == 0)
        def _():
            run(m0_hbm)

        @pl.when(c == 1)
        def _():
            run(m1_hbm)

        plsc.subcore_barrier()

        @pl.when(jnp.logical_and(c == 0, s < _ZTILES))
        def _():
            pltpu.sync_copy(acc_sh.at[pl.ds(s * _ZROWS, _ZROWS)],
                            a0_hbm.at[pl.ds(s * _ZROWS, _ZROWS)])

        @pl.when(jnp.logical_and(c == 1, s < _ZTILES))
        def _():
            pltpu.sync_copy(acc_sh.at[pl.ds(s * _ZROWS, _ZROWS)],
                            a1_hbm.at[pl.ds(s * _ZROWS, _ZROWS)])

    def call(m0, m1, dst3d, z0, z1):
        k = functools.partial(
            pl.kernel,
            mesh=plsc.VectorSubcoreMesh(core_axis_name="c",
                                        subcore_axis_name="s"),
            out_type=[
                jax.ShapeDtypeStruct((N_NODES, _HF), jnp.float32),
                jax.ShapeDtypeStruct((N_NODES, _HF), jnp.float32),
            ],
            scratch_types=[
                pltpu.VMEM((sit, sch), jnp.int32),
                pltpu.VMEM((sch, _HF), jnp.float32),
                pltpu.VMEM((sch, _HF), jnp.float32),
                pltpu.VMEM_SHARED((N_NODES, _HF), jnp.float32),
                pltpu.SemaphoreType.DMA,
                pltpu.SemaphoreType.DMA,
                pltpu.SemaphoreType.DMA,
                pltpu.SemaphoreType.DMA,
            ],
        )(_scatter_kernel)
        return k(m0, m1, dst3d, z0, z1)

    return call


_SCH = 40
_scatter_add_half = _make_scatter_add(_EHALF, _SCH)


def _mlp_body(x_ref, a0_ref, a1_ref, wffw_ref, bffw_ref, wproj_ref,
              bproj_ref, o_ref):
    h = x_ref[...] + jnp.concatenate([a0_ref[...], a1_ref[...]], axis=1)
    h = jnp.maximum(
        jnp.dot(h, wffw_ref[...], preferred_element_type=jnp.float32)
        + bffw_ref[...], 0.0)
    o_ref[...] = (jnp.dot(h, wproj_ref[...], preferred_element_type=jnp.float32)
                  + bproj_ref[...])


def _out_mlp(x, a0, a1, wffw, bffw, wproj, bproj):
    grid = (N_NODES // _ROWS_BLK,)
    return pl.pallas_call(
        _mlp_body,
        grid=grid,
        in_specs=[
            pl.BlockSpec((_ROWS_BLK, F_DIM), lambda i: (i, 0)),
            pl.BlockSpec((_ROWS_BLK, _HF), lambda i: (i, 0)),
            pl.BlockSpec((_ROWS_BLK, _HF), lambda i: (i, 0)),
            pl.BlockSpec((F_DIM, F_DIM), lambda i: (0, 0)),
            pl.BlockSpec((1, F_DIM), lambda i: (0, 0)),
            pl.BlockSpec((F_DIM, F_DIM), lambda i: (0, 0)),
            pl.BlockSpec((1, F_DIM), lambda i: (0, 0)),
        ],
        out_specs=pl.BlockSpec((_ROWS_BLK, F_DIM), lambda i: (i, 0)),
        out_shape=jax.ShapeDtypeStruct((N_NODES, F_DIM), jnp.float32),
    )(x, a0, a1, wffw, bffw, wproj, bproj)


def kernel(x, edge_index, edge_attr, Wf, bf, Ws, bs, Wffw, bffw, Wproj, bproj):
    src = edge_index[0]
    dst = edge_index[1]
    d_tab, s_tab = _node_tables(x, Wf[:F_DIM], Ws[:F_DIM],
                                Wf[F_DIM:2 * F_DIM], Ws[F_DIM:2 * F_DIM])
    weg, wec = Wf[2 * F_DIM:], Ws[2 * F_DIM:]
    bg, bc = bf.reshape(1, F_DIM), bs.reshape(1, F_DIM)
    rd, rs = _edge_gather_full(d_tab, s_tab, dst, src)
    m0a, m1a = _edge_messages_half(rd, rs, edge_attr, weg, wec, bg, bc, 0)
    m0b, m1b = _edge_messages_half(rd, rs, edge_attr, weg, wec, bg, bc, 1)
    sit = (_EHALF // NS) // _SCH
    dst3d_a = dst[:_EHALF].reshape(NS, sit, _SCH)
    dst3d_b = dst[_EHALF:].reshape(NS, sit, _SCH)
    zeros_full = jnp.zeros((N_NODES, _HF), jnp.float32)
    a0p, a1p = _scatter_add_half(m0a, m1a, dst3d_a, zeros_full, zeros_full)
    a0, a1 = _scatter_add_half(m0b, m1b, dst3d_b, a0p, a1p)
    return _out_mlp(x, a0, a1, bffw=bffw.reshape(1, F_DIM), wffw=Wffw,
                    wproj=Wproj, bproj=bproj.reshape(1, F_DIM))


# speedup vs baseline: 1.0317x; 1.0317x over previous
import functools

import jax
import jax.numpy as jnp
from jax import lax
from jax.experimental import pallas as pl
from jax.experimental.pallas import tpu as pltpu
from jax.experimental.pallas import tpu_sc as plsc

N_NODES = 10000
N_EDGES = 160000
F_DIM = 256
DE_DIM = 16

NC = 2
NS = 16
NW = NC * NS


_ROWS_BLK = 1000


def _rne16(x):
    b = jax.lax.bitcast_convert_type(x, jnp.int32)
    r = b + 0x7FFF + jax.lax.shift_right_logical(b, 16) % 2
    return jax.lax.shift_right_logical(r, 16)


def _pack2(gate_f32, core_f32):
    return _rne16(gate_f32) | (_rne16(core_f32) << 16)


def _unpack_lo(w):
    return jax.lax.bitcast_convert_type(w << 16, jnp.float32)


def _unpack_hi(w):
    return jax.lax.bitcast_convert_type(w & jnp.int32(-65536), jnp.float32)


def _tables_body(x_ref, wfd_ref, wsd_ref, wfs_ref, wss_ref, d_ref, s_ref):
    xb = x_ref[...]
    d_ref[...] = _pack2(
        jnp.dot(xb, wfd_ref[...], preferred_element_type=jnp.float32),
        jnp.dot(xb, wsd_ref[...], preferred_element_type=jnp.float32))
    s_ref[...] = _pack2(
        jnp.dot(xb, wfs_ref[...], preferred_element_type=jnp.float32),
        jnp.dot(xb, wss_ref[...], preferred_element_type=jnp.float32))


def _node_tables(x, wfd, wsd, wfs, wss):
    grid = (N_NODES // _ROWS_BLK,)
    wspec = pl.BlockSpec((F_DIM, F_DIM), lambda i: (0, 0))
    return pl.pallas_call(
        _tables_body,
        grid=grid,
        in_specs=[
            pl.BlockSpec((_ROWS_BLK, F_DIM), lambda i: (i, 0)),
            wspec, wspec, wspec, wspec,
        ],
        out_specs=[
            pl.BlockSpec((_ROWS_BLK, F_DIM), lambda i: (i, 0)),
            pl.BlockSpec((_ROWS_BLK, F_DIM), lambda i: (i, 0)),
        ],
        out_shape=[
            jax.ShapeDtypeStruct((N_NODES, F_DIM), jnp.int32),
            jax.ShapeDtypeStruct((N_NODES, F_DIM), jnp.int32),
        ],
    )(x, wfd, wsd, wfs, wss)


def _make_edge_gather(n_edges, gch):
    ew = n_edges // NS
    git = ew // gch

    def _gather_kernel(d_hbm, s_hbm, dst_hbm, src_hbm, rd_hbm, rs_hbm,
                       idxv, buf0, buf1, sem0, sem1):
        c = lax.axis_index("c")
        s = lax.axis_index("s")
        base = s * ew
        bufs = (buf0, buf1)
        sems = (sem0, sem1)

        def run(tab_hbm, idx_hbm, out_hbm):
            pltpu.sync_copy(idx_hbm.at[pl.ds(base, ew)], idxv)

            def start(b, i):
                pltpu.async_copy(tab_hbm.at[idxv.at[pl.ds(i * gch, gch)]],
                                 bufs[b], sems[b])

            def finish(b, i):
                pltpu.make_async_copy(
                    tab_hbm.at[idxv.at[pl.ds(i * gch, gch)]],
                    bufs[b], sems[b]).wait()
                pltpu.sync_copy(bufs[b],
                                out_hbm.at[pl.ds(base + i * gch, gch)])

            start(0, 0)
            start(1, 1)

            def body(g, carry):
                for b in range(2):
                    i = 2 * g + b
                    finish(b, i)

                    @pl.when(i + 2 < git)
                    def _():
                        start(b, i + 2)
                return carry

            lax.fori_loop(0, git // 2, body, 0)
            if git % 2:
                finish(0, git - 1)

        @pl.when(c == 0)
        def _():
            run(d_hbm, dst_hbm, rd_hbm)

        @pl.when(c == 1)
        def _():
            run(s_hbm, src_hbm, rs_hbm)

    def call(d, s, dst, src):
        k = functools.partial(
            pl.kernel,
            mesh=plsc.VectorSubcoreMesh(core_axis_name="c",
                                        subcore_axis_name="s"),
            out_type=[
                jax.ShapeDtypeStruct((n_edges, F_DIM), jnp.int32),
                jax.ShapeDtypeStruct((n_edges, F_DIM), jnp.int32),
            ],
            scratch_types=[
                pltpu.VMEM((ew,), jnp.int32),
                pltpu.VMEM((gch, F_DIM), jnp.int32),
                pltpu.VMEM((gch, F_DIM), jnp.int32),
                pltpu.SemaphoreType.DMA,
                pltpu.SemaphoreType.DMA,
            ],
        )(_gather_kernel)
        return k(d, s, dst, src)

    return call


_EHALF = N_EDGES // 2
_edge_gather_half = _make_edge_gather(_EHALF, 200)


_EDGE_BLK = 2000


def _msg_body(rd_ref, rs_ref, ea_ref, weg_ref, wec_ref, bg_ref, bc_ref,
              m0_ref, m1_ref):
    wd = rd_ref[...]
    ws = rs_ref[...]
    ea = ea_ref[...]
    g = (_unpack_lo(wd) + _unpack_lo(ws)
         + jnp.dot(ea, weg_ref[...], preferred_element_type=jnp.float32)
         + bg_ref[...])
    c = (_unpack_hi(wd) + _unpack_hi(ws)
         + jnp.dot(ea, wec_ref[...], preferred_element_type=jnp.float32)
         + bc_ref[...])
    gate = 1.0 / (1.0 + jnp.exp(-g))
    sp = jnp.maximum(c, 0.0) + jnp.log(1.0 + jnp.exp(-jnp.abs(c)))
    msg = gate * sp
    m0_ref[...] = msg[:, : F_DIM // 2]
    m1_ref[...] = msg[:, F_DIM // 2:]


def _edge_messages_half(rd, rs, ea, weg, wec, bg, bc, half):
    grid = (_EHALF // _EDGE_BLK,)
    off = half * (_EHALF // _EDGE_BLK)
    wspec = pl.BlockSpec((DE_DIM, F_DIM), lambda i: (0, 0))
    bspec = pl.BlockSpec((1, F_DIM), lambda i: (0, 0))
    return pl.pallas_call(
        _msg_body,
        grid=grid,
        in_specs=[
            pl.BlockSpec((_EDGE_BLK, F_DIM), lambda i: (i, 0)),
            pl.BlockSpec((_EDGE_BLK, F_DIM), lambda i: (i, 0)),
            pl.BlockSpec((_EDGE_BLK, DE_DIM), lambda i: (i + off, 0)),
            wspec, wspec, bspec, bspec,
        ],
        out_specs=[
            pl.BlockSpec((_EDGE_BLK, F_DIM // 2), lambda i: (i, 0)),
            pl.BlockSpec((_EDGE_BLK, F_DIM // 2), lambda i: (i, 0)),
        ],
        out_shape=[
            jax.ShapeDtypeStruct((_EHALF, F_DIM // 2), jnp.float32),
            jax.ShapeDtypeStruct((_EHALF, F_DIM // 2), jnp.float32),
        ],
    )(rd, rs, ea, weg, wec, bg, bc)


_ZROWS = 1000
_ZTILES = N_NODES // _ZROWS
_HF = F_DIM // 2


def _make_scatter_add(n_edges, sch):
    et = n_edges // NS
    sit = et // sch

    def _scatter_kernel(m0_hbm, m1_hbm, dst3d_hbm, z0_hbm, z1_hbm,
                        a0_hbm, a1_hbm,
                        idxv, mbuf0, mbuf1, acc_sh,
                        seml0, seml1, semc0, semc1):
        c = lax.axis_index("c")
        s = lax.axis_index("s")

        @pl.when(jnp.logical_and(c == 0, s < _ZTILES))
        def _():
            pltpu.sync_copy(z0_hbm.at[pl.ds(s * _ZROWS, _ZROWS)],
                            acc_sh.at[pl.ds(s * _ZROWS, _ZROWS)])

        @pl.when(jnp.logical_and(c == 1, s < _ZTILES))
        def _():
            pltpu.sync_copy(z1_hbm.at[pl.ds(s * _ZROWS, _ZROWS)],
                            acc_sh.at[pl.ds(s * _ZROWS, _ZROWS)])

        plsc.subcore_barrier()
        pltpu.sync_copy(dst3d_hbm.at[s], idxv)

        mbufs = (mbuf0, mbuf1)
        semls = (seml0, seml1)
        semcs = (semc0, semc1)

        def run(m_hbm):
            def startload(b, i):
                pltpu.async_copy(m_hbm.at[pl.ds(s * et + i * sch, sch)],
                                 mbufs[b], semls[b])

            def finish(b, i):
                pltpu.make_async_copy(
                    m_hbm.at[pl.ds(s * et + i * sch, sch)],
                    mbufs[b], semls[b]).wait()
                pltpu.async_copy(mbufs[b], acc_sh.at[idxv.at[i]], semcs[b],
                                 add=True)
                pltpu.make_async_copy(mbufs[b], acc_sh.at[idxv.at[i]],
                                      semcs[b]).wait()

            startload(0, 0)
            startload(1, 1)

            def body(g, carry):
                for b in range(2):
                    i = 2 * g + b
                    finish(b, i)

                    @pl.when(i + 2 < sit)
                    def _():
                        startload(b, i + 2)
                return carry

            lax.fori_loop(0, sit // 2, body, 0)
            if sit % 2:
                finish(0, sit - 1)

        @pl.when(c == 0)
        def _():
            run(m0_hbm)

        @pl.when(c == 1)
        def _():
            run(m1_hbm)

        plsc.subcore_barrier()

        @pl.when(jnp.logical_and(c == 0, s < _ZTILES))
        def _():
            pltpu.sync_copy(acc_sh.at[pl.ds(s * _ZROWS, _ZROWS)],
                            a0_hbm.at[pl.ds(s * _ZROWS, _ZROWS)])

        @pl.when(jnp.logical_and(c == 1, s < _ZTILES))
        def _():
            pltpu.sync_copy(acc_sh.at[pl.ds(s * _ZROWS, _ZROWS)],
                            a1_hbm.at[pl.ds(s * _ZROWS, _ZROWS)])

    def call(m0, m1, dst3d, z0, z1):
        k = functools.partial(
            pl.kernel,
            mesh=plsc.VectorSubcoreMesh(core_axis_name="c",
                                        subcore_axis_name="s"),
            out_type=[
                jax.ShapeDtypeStruct((N_NODES, _HF), jnp.float32),
                jax.ShapeDtypeStruct((N_NODES, _HF), jnp.float32),
            ],
            scratch_types=[
                pltpu.VMEM((sit, sch), jnp.int32),
                pltpu.VMEM((sch, _HF), jnp.float32),
                pltpu.VMEM((sch, _HF), jnp.float32),
                pltpu.VMEM_SHARED((N_NODES, _HF), jnp.float32),
                pltpu.SemaphoreType.DMA,
                pltpu.SemaphoreType.DMA,
                pltpu.SemaphoreType.DMA,
                pltpu.SemaphoreType.DMA,
            ],
        )(_scatter_kernel)
        return k(m0, m1, dst3d, z0, z1)

    return call


_SCH = 40
_scatter_add_half = _make_scatter_add(_EHALF, _SCH)


def _mlp_body(x_ref, a0_ref, a1_ref, wffw_ref, bffw_ref, wproj_ref,
              bproj_ref, o_ref):
    h = x_ref[...] + jnp.concatenate([a0_ref[...], a1_ref[...]], axis=1)
    h = jnp.maximum(
        jnp.dot(h, wffw_ref[...], preferred_element_type=jnp.float32)
        + bffw_ref[...], 0.0)
    o_ref[...] = (jnp.dot(h, wproj_ref[...], preferred_element_type=jnp.float32)
                  + bproj_ref[...])


def _out_mlp(x, a0, a1, wffw, bffw, wproj, bproj):
    grid = (N_NODES // _ROWS_BLK,)
    return pl.pallas_call(
        _mlp_body,
        grid=grid,
        in_specs=[
            pl.BlockSpec((_ROWS_BLK, F_DIM), lambda i: (i, 0)),
            pl.BlockSpec((_ROWS_BLK, _HF), lambda i: (i, 0)),
            pl.BlockSpec((_ROWS_BLK, _HF), lambda i: (i, 0)),
            pl.BlockSpec((F_DIM, F_DIM), lambda i: (0, 0)),
            pl.BlockSpec((1, F_DIM), lambda i: (0, 0)),
            pl.BlockSpec((F_DIM, F_DIM), lambda i: (0, 0)),
            pl.BlockSpec((1, F_DIM), lambda i: (0, 0)),
        ],
        out_specs=pl.BlockSpec((_ROWS_BLK, F_DIM), lambda i: (i, 0)),
        out_shape=jax.ShapeDtypeStruct((N_NODES, F_DIM), jnp.float32),
    )(x, a0, a1, wffw, bffw, wproj, bproj)


def kernel(x, edge_index, edge_attr, Wf, bf, Ws, bs, Wffw, bffw, Wproj, bproj):
    src = edge_index[0]
    dst = edge_index[1]
    d_tab, s_tab = _node_tables(x, Wf[:F_DIM], Ws[:F_DIM],
                                Wf[F_DIM:2 * F_DIM], Ws[F_DIM:2 * F_DIM])
    weg, wec = Wf[2 * F_DIM:], Ws[2 * F_DIM:]
    bg, bc = bf.reshape(1, F_DIM), bs.reshape(1, F_DIM)
    rd0, rs0 = _edge_gather_half(d_tab, s_tab, dst[:_EHALF], src[:_EHALF])
    rd1, rs1 = _edge_gather_half(d_tab, s_tab, dst[_EHALF:], src[_EHALF:])
    m0a, m1a = _edge_messages_half(rd0, rs0, edge_attr, weg, wec, bg, bc, 0)
    m0b, m1b = _edge_messages_half(rd1, rs1, edge_attr, weg, wec, bg, bc, 1)
    sit = (_EHALF // NS) // _SCH
    dst3d_a = dst[:_EHALF].reshape(NS, sit, _SCH)
    dst3d_b = dst[_EHALF:].reshape(NS, sit, _SCH)
    zeros_full = jnp.zeros((N_NODES, _HF), jnp.float32)
    a0p, a1p = _scatter_add_half(m0a, m1a, dst3d_a, zeros_full, zeros_full)
    a0, a1 = _scatter_add_half(m0b, m1b, dst3d_b, a0p, a1p)
    return _out_mlp(x, a0, a1, bffw=bffw.reshape(1, F_DIM), wffw=Wffw,
                    wproj=Wproj, bproj=bproj.reshape(1, F_DIM))
